# baseline (speedup 1.0000x reference)
import jax
import jax.numpy as jnp
import numpy as np
from jax import lax
from jax.experimental import pallas as pl

NUM_TOKENS = 16384
INPUT_DIM = 2048
NUM_EXPERTS = 64
BLOCK_T = 1024

_KS0 = np.uint32(0)
_KS1 = np.uint32(1)
_KS2 = np.uint32(0 ^ 1 ^ 0x1BD11BDA)
_ROT_A = (13, 15, 26, 6)
_ROT_B = (17, 29, 16, 24)


def _rotl(x, r):
    return (x << np.uint32(r)) | (x >> np.uint32(32 - r))


def _rounds(x0, x1, rots):
    for r in rots:
        x0 = x0 + x1
        x1 = _rotl(x1, r)
        x1 = x0 ^ x1
    return x0, x1


def _noise_block(flat_base, shape):
    c_lo = (jnp.uint32(flat_base)
            + lax.broadcasted_iota(jnp.uint32, shape, 0) * np.uint32(shape[1])
            + lax.broadcasted_iota(jnp.uint32, shape, 1))
    x0 = jnp.zeros(shape, jnp.uint32) + _KS0
    x1 = c_lo + _KS1
    x0, x1 = _rounds(x0, x1, _ROT_A)
    x0 = x0 + _KS1
    x1 = x1 + _KS2 + np.uint32(1)
    x0, x1 = _rounds(x0, x1, _ROT_B)
    x0 = x0 + _KS2
    x1 = x1 + _KS0 + np.uint32(2)
    x0, x1 = _rounds(x0, x1, _ROT_A)
    x0 = x0 + _KS0
    x1 = x1 + _KS1 + np.uint32(3)
    x0, x1 = _rounds(x0, x1, _ROT_B)
    x0 = x0 + _KS1
    x1 = x1 + _KS2 + np.uint32(4)
    x0, x1 = _rounds(x0, x1, _ROT_A)
    x0 = x0 + _KS2
    x1 = x1 + _KS0 + np.uint32(5)
    bits = x0 ^ x1
    fb = (bits >> np.uint32(9)) | np.uint32(0x3F800000)
    f = lax.bitcast_convert_type(fb, jnp.float32) - np.float32(1.0)
    lo = np.nextafter(np.float32(-1.0), np.float32(0.0), dtype=np.float32)
    hi = np.float32(1.0)
    u = lax.max(jnp.float32(lo), f * (hi - lo) + lo)
    return np.float32(np.sqrt(2.0)) * lax.erf_inv(u)


def _gating_body(x_ref, w_ref, b_ref, o_ref):
    logits = lax.dot_general(
        x_ref[...], w_ref[...],
        dimension_numbers=(((1,), (1,)), ((), ())),
        preferred_element_type=jnp.float32,
    )
    i = pl.program_id(0)
    noise = _noise_block(i * (BLOCK_T * NUM_EXPERTS), (BLOCK_T, NUM_EXPERTS))
    noisy = logits + b_ref[...] + noise

    col = lax.broadcasted_iota(jnp.int32, noisy.shape, 1)
    m1 = jnp.max(noisy, axis=-1, keepdims=True)
    i1 = jnp.min(jnp.where(noisy == m1, col, NUM_EXPERTS), axis=-1,
                 keepdims=True)
    is1 = col == i1
    masked = jnp.where(is1, -jnp.inf, noisy)
    m2 = jnp.max(masked, axis=-1, keepdims=True)
    i2 = jnp.min(jnp.where(masked == m2, col, NUM_EXPERTS), axis=-1,
                 keepdims=True)
    is2 = col == i2

    t = jnp.exp(m2 - m1)
    p1 = 1.0 / (1.0 + t)
    o_ref[...] = jnp.where(is1, p1, 0.0) + jnp.where(is2, t * p1, 0.0)


@jax.jit
def kernel(x, W, b):
    n_tokens = x.shape[0]
    grid = (n_tokens // BLOCK_T,)
    return pl.pallas_call(
        _gating_body,
        grid=grid,
        in_specs=[
            pl.BlockSpec((BLOCK_T, INPUT_DIM), lambda i: (i, 0)),
            pl.BlockSpec((NUM_EXPERTS, INPUT_DIM), lambda i: (0, 0)),
            pl.BlockSpec((1, NUM_EXPERTS), lambda i: (0, 0)),
        ],
        out_specs=pl.BlockSpec((BLOCK_T, NUM_EXPERTS), lambda i: (i, 0)),
        out_shape=jax.ShapeDtypeStruct((n_tokens, NUM_EXPERTS), jnp.float32),
    )(x, W, b.reshape(1, NUM_EXPERTS))
